# baseline (speedup 1.0000x reference)
import functools

import jax
import jax.numpy as jnp
import numpy as np
from jax import lax
from jax.experimental import pallas as pl
from jax.experimental.pallas import tpu as pltpu
from jax.experimental.pallas import tpu_sc as plsc

VOCAB = 256
DIM = 64
LANES = 16

NC = 2
NS = 16
NW = NC * NS
REP = 16

CHUNK = 32
NBUF = 8

_U = np.arange(VOCAB // 2)
PERM = np.concatenate([
    (_U // LANES) * 2 * LANES + _U % LANES,
    (_U // LANES) * 2 * LANES + LANES + _U % LANES,
]).astype(np.int32)


def _table_body(embed_t_ref, w_t_ref, b_ref, t_ref):
    t = (
        lax.dot_general(
            embed_t_ref[...],
            w_t_ref[...],
            (((0,), (0,)), ((), ())),
            preferred_element_type=jnp.float32,
        )
        + b_ref[...]
    )
    lo = lax.bitcast_convert_type(t[:, : VOCAB // 2], jnp.int32)
    hi = lax.bitcast_convert_type(t[:, VOCAB // 2 :], jnp.int32)
    word = lax.shift_right_logical(lo + 0x8000, 16) | (
        (hi + 0x8000) & jnp.int32(-65536)
    )
    t_ref[...] = jnp.broadcast_to(word[None], t_ref.shape)


def _make_table(embed, W, b):
    w_t = jnp.take(W, jnp.asarray(PERM), axis=0).T
    b_p = jnp.take(b, jnp.asarray(PERM)).reshape(1, VOCAB)
    return pl.pallas_call(
        _table_body,
        out_shape=jax.ShapeDtypeStruct((REP, VOCAB, VOCAB // 2), jnp.int32),
    )(embed.T, w_t, b_p)


def _make_gather(n_tokens):
    assert n_tokens % (NW * CHUNK) == 0
    bpw = n_tokens // NW
    nchunk = bpw // CHUNK
    assert nchunk % NBUF == 0

    mesh = plsc.VectorSubcoreMesh(core_axis_name="c", subcore_axis_name="s")

    @functools.partial(
        pl.kernel,
        mesh=mesh,
        out_type=jax.ShapeDtypeStruct((n_tokens, VOCAB), jnp.int32),
        scratch_types=[
            pltpu.VMEM((bpw,), jnp.int32),
        ]
        + [pltpu.VMEM((CHUNK, VOCAB // 2), jnp.int32) for _ in range(NBUF)]
        + [pltpu.VMEM((CHUNK, VOCAB), jnp.int32) for _ in range(NBUF)]
        + [pltpu.SemaphoreType.DMA for _ in range(2 * NBUF)],
    )
    def gather(table_hbm, idx_hbm, out_hbm, idx_v, *rest):
        gbufs = rest[:NBUF]
        fbufs = rest[NBUF : 2 * NBUF]
        gsems = rest[2 * NBUF : 3 * NBUF]
        wsems = rest[3 * NBUF :]
        wid = lax.axis_index("s") * NC + lax.axis_index("c")
        base = wid * bpw
        tpr = idx_hbm.shape[1] // bpw
        pltpu.sync_copy(
            idx_hbm.at[wid // tpr, pl.ds(lax.rem(wid, tpr) * bpw, bpw)], idx_v
        )
        tbl = table_hbm.at[lax.rem(wid, REP)]

        def gcopy(j, i):
            return pltpu.make_async_copy(
                tbl.at[idx_v.at[pl.ds(j * CHUNK, CHUNK)]], gbufs[i], gsems[i]
            )

        def wcopy(j, i):
            return pltpu.make_async_copy(
                fbufs[i], out_hbm.at[pl.ds(base + j * CHUNK, CHUNK)], wsems[i]
            )

        def convert(i):
            gb, fb = gbufs[i], fbufs[i]

            shift = jnp.full((LANES,), 65536, jnp.int32)
            mask = jnp.full((LANES,), -65536, jnp.int32)

            def row(r, _):
                for g in range(VOCAB // (2 * LANES)):
                    w = gb[r, pl.ds(g * LANES, LANES)]
                    fb[r, pl.ds(g * 2 * LANES, LANES)] = w * shift
                    fb[r, pl.ds(g * 2 * LANES + LANES, LANES)] = w & mask
                return 0

            lax.fori_loop(0, CHUNK, row, 0, unroll=False)

        for i in range(NBUF - 1):
            gcopy(i, i).start()

        def outer(g, _):
            for i in range(NBUF):
                j = g * NBUF + i
                gcopy(j, i).wait()

                @pl.when(g >= 1)
                def _():
                    wcopy(j - NBUF, i).wait()

                convert(i)
                wcopy(j, i).start()
                nx = j + NBUF - 1
                ib = (i + NBUF - 1) % NBUF
                if i == 0:
                    gcopy(nx, ib).start()
                else:
                    @pl.when(nx < nchunk)
                    def _():
                        gcopy(nx, ib).start()
            return 0

        lax.fori_loop(0, nchunk // NBUF, outer, 0, unroll=False)
        for i in range(NBUF):
            wcopy(nchunk - NBUF + i, i).wait()

    return gather


def kernel(x, embed, W, b):
    batch, seq = x.shape
    n_tokens = batch * seq
    table = _make_table(embed, W, b)
    out = _make_gather(n_tokens)(table, x)
    out = lax.bitcast_convert_type(out, jnp.float32)
    return out.reshape(batch, seq, VOCAB)


# speedup vs baseline: 1.9474x; 1.9474x over previous
import functools

import jax
import jax.numpy as jnp
from jax import lax
from jax.experimental import pallas as pl
from jax.experimental.pallas import tpu as pltpu
from jax.experimental.pallas import tpu_sc as plsc

VOCAB = 256
DIM = 64

NC = 2
NS = 16
NW = NC * NS
REP = 16

CHUNK = 32
NBUF = 8


def _table_body(embed_t_ref, w_t_ref, b_ref, t_ref):
    t = (
        lax.dot_general(
            embed_t_ref[...],
            w_t_ref[...],
            (((0,), (0,)), ((), ())),
            preferred_element_type=jnp.float32,
        )
        + b_ref[...]
    )
    t_ref[...] = jnp.broadcast_to(t[None], t_ref.shape)


def _make_table(embed, W, b):
    return pl.pallas_call(
        _table_body,
        out_shape=jax.ShapeDtypeStruct((REP, VOCAB, VOCAB), jnp.float32),
    )(embed.T, W.T, b.reshape(1, VOCAB))


def _make_gather(n_tokens):
    assert n_tokens % (NW * CHUNK) == 0
    bpw = n_tokens // NW
    nchunk = bpw // CHUNK
    assert nchunk % NBUF == 0

    mesh = plsc.VectorSubcoreMesh(core_axis_name="c", subcore_axis_name="s")

    @functools.partial(
        pl.kernel,
        mesh=mesh,
        out_type=jax.ShapeDtypeStruct((n_tokens, VOCAB), jnp.float32),
        scratch_types=[
            pltpu.VMEM((bpw,), jnp.int32),
        ]
        + [pltpu.VMEM((CHUNK, VOCAB), jnp.float32) for _ in range(NBUF)]
        + [pltpu.SemaphoreType.DMA for _ in range(2 * NBUF)],
    )
    def gather(table_hbm, idx_hbm, out_hbm, idx_v, *rest):
        bufs = rest[:NBUF]
        gsems = rest[NBUF : 2 * NBUF]
        wsems = rest[2 * NBUF :]
        wid = lax.axis_index("s") * NC + lax.axis_index("c")
        base = wid * bpw
        tpr = idx_hbm.shape[1] // bpw
        pltpu.sync_copy(
            idx_hbm.at[wid // tpr, pl.ds(lax.rem(wid, tpr) * bpw, bpw)], idx_v
        )
        tbl = table_hbm.at[lax.rem(wid, REP)]

        def gcopy(j, i):
            return pltpu.make_async_copy(
                tbl.at[idx_v.at[pl.ds(j * CHUNK, CHUNK)]], bufs[i], gsems[i]
            )

        def wcopy(j, i):
            return pltpu.make_async_copy(
                bufs[i], out_hbm.at[pl.ds(base + j * CHUNK, CHUNK)], wsems[i]
            )

        for i in range(NBUF - 1):
            gcopy(i, i).start()

        def outer(g, _):
            for i in range(NBUF):
                j = g * NBUF + i
                gcopy(j, i).wait()
                wcopy(j, i).start()
                nx = j + NBUF - 1
                ib = (i + NBUF - 1) % NBUF
                if i == 0:
                    @pl.when(g >= 1)
                    def _():
                        wcopy(nx - NBUF, ib).wait()

                    gcopy(nx, ib).start()
                else:
                    @pl.when(nx < nchunk)
                    def _():
                        wcopy(nx - NBUF, ib).wait()
                        gcopy(nx, ib).start()
            return 0

        lax.fori_loop(0, nchunk // NBUF, outer, 0, unroll=False)
        for i in range(NBUF):
            wcopy(nchunk - NBUF + i, i).wait()

    return gather


def kernel(x, embed, W, b):
    batch, seq = x.shape
    n_tokens = batch * seq
    table = _make_table(embed, W, b)
    out = _make_gather(n_tokens)(table, x)
    return out.reshape(batch, seq, VOCAB)
